# baseline (speedup 1.0000x reference)
import functools
import math

import jax
import jax.numpy as jnp
from jax import lax
from jax.experimental import pallas as pl
from jax.experimental.pallas import tpu as pltpu
from jax.experimental.pallas import tpu_sc as plsc

D_MODEL = 64
SCALE = math.sqrt(D_MODEL)

NUM_CORES = 2
NUM_SUBCORES = 16
NUM_WORKERS = NUM_CORES * NUM_SUBCORES

CHUNK = 128
NBUF = 2


def _emb_body(x_hbm, table_hbm, out_hbm, idx_v, in0, in1, ou0, ou1,
              gs0, gs1, os0, os1, *, per_w, n_chunks):
    wid = lax.axis_index("s") * NUM_CORES + lax.axis_index("c")
    base = wid * per_w
    ins, outs = (in0, in1), (ou0, ou1)
    gsems, osems = (gs0, gs1), (os0, os1)

    pltpu.sync_copy(x_hbm.at[pl.ds(base, per_w)], idx_v)

    def idx_slice(ci):
        off = pl.multiple_of(ci * CHUNK, CHUNK)
        return idx_v.at[pl.ds(off, CHUNK)]

    for b in range(NBUF):
        pltpu.async_copy(table_hbm.at[idx_slice(b)], ins[b], gsems[b])

    def group(g, carry):
        for b in range(NBUF):
            ci = g * NBUF + b
            off = base + ci * CHUNK
            pltpu.make_async_copy(table_hbm.at[idx_slice(ci)], ins[b],
                                  gsems[b]).wait()

            @pl.when(g > 0)
            def _():
                pltpu.make_async_copy(
                    outs[b], out_hbm.at[pl.ds(base, CHUNK)], osems[b]).wait()

            def row(j, c):
                for k in range(D_MODEL // 16):
                    sl = pl.ds(k * 16, 16)
                    outs[b][j, sl] = ins[b][j, sl] * SCALE
                return c

            lax.fori_loop(0, CHUNK, row, 0, unroll=4)
            pltpu.async_copy(outs[b], out_hbm.at[pl.ds(off, CHUNK)], osems[b])

            @pl.when(ci < n_chunks - NBUF)
            def _():
                pltpu.async_copy(table_hbm.at[idx_slice(ci + NBUF)], ins[b],
                                 gsems[b])
        return carry

    lax.fori_loop(0, n_chunks // NBUF, group, 0)

    for b in range(NBUF):
        pltpu.make_async_copy(outs[b], out_hbm.at[pl.ds(base, CHUNK)],
                              osems[b]).wait()


def kernel(x, table):
    b0, b1 = x.shape
    n_total = b0 * b1
    assert n_total % (NUM_WORKERS * CHUNK * NBUF) == 0
    per_w = n_total // NUM_WORKERS
    n_chunks = per_w // CHUNK

    mesh = plsc.VectorSubcoreMesh(core_axis_name="c", subcore_axis_name="s")
    emb = functools.partial(
        pl.kernel,
        mesh=mesh,
        out_type=jax.ShapeDtypeStruct((n_total, D_MODEL), jnp.float32),
        scratch_types=[
            pltpu.VMEM((per_w,), jnp.int32),
            pltpu.VMEM((CHUNK, D_MODEL), jnp.float32),
            pltpu.VMEM((CHUNK, D_MODEL), jnp.float32),
            pltpu.VMEM((CHUNK, D_MODEL), jnp.float32),
            pltpu.VMEM((CHUNK, D_MODEL), jnp.float32),
            pltpu.SemaphoreType.DMA,
            pltpu.SemaphoreType.DMA,
            pltpu.SemaphoreType.DMA,
            pltpu.SemaphoreType.DMA,
        ],
        compiler_params=pltpu.CompilerParams(use_tc_tiling_on_sc=False),
    )(functools.partial(_emb_body, per_w=per_w, n_chunks=n_chunks))

    out = emb(x.reshape(n_total), table)
    return out.reshape(b0, b1, D_MODEL)


# speedup vs baseline: 1.6488x; 1.6488x over previous
import functools
import math

import jax
import jax.numpy as jnp
from jax import lax
from jax.experimental import pallas as pl
from jax.experimental.pallas import tpu as pltpu
from jax.experimental.pallas import tpu_sc as plsc

D_MODEL = 64
SCALE = math.sqrt(D_MODEL)

NUM_CORES = 2
NUM_SUBCORES = 16
NUM_WORKERS = NUM_CORES * NUM_SUBCORES

CHUNK = 128
NBUF = 2


def _emb_body(x_hbm, table_hbm, out_hbm, idx_v, in0, in1, ou0, ou1,
              gs0, gs1, os0, os1, *, per_w, n_chunks):
    wid = lax.axis_index("s") * NUM_CORES + lax.axis_index("c")
    base = wid * per_w
    ins, outs = (in0, in1), (ou0, ou1)
    gsems, osems = (gs0, gs1), (os0, os1)

    pltpu.sync_copy(x_hbm.at[pl.ds(base, per_w)], idx_v)

    def idx_slice(ci):
        off = pl.multiple_of(ci * CHUNK, CHUNK)
        return idx_v.at[pl.ds(off, CHUNK)]

    for b in range(NBUF):
        pltpu.async_copy(table_hbm.at[idx_slice(b)], ins[b], gsems[b])

    def group(g, carry):
        for b in range(NBUF):
            ci = g * NBUF + b
            off = base + ci * CHUNK
            pltpu.make_async_copy(table_hbm.at[idx_slice(ci)], ins[b],
                                  gsems[b]).wait()

            @pl.when(g > 0)
            def _():
                pltpu.make_async_copy(
                    outs[b], out_hbm.at[pl.ds(base, CHUNK), pl.ds(0, D_MODEL)],
                    osems[b]).wait()

            @plsc.parallel_loop(0, CHUNK, unroll=8)
            def _(j):
                for k in range(D_MODEL // 16):
                    sl = pl.ds(k * 16, 16)
                    outs[b][j, sl] = ins[b][j, sl] * SCALE

            pltpu.async_copy(
                outs[b], out_hbm.at[pl.ds(off, CHUNK), pl.ds(0, D_MODEL)],
                osems[b])

            @pl.when(ci < n_chunks - NBUF)
            def _():
                pltpu.async_copy(table_hbm.at[idx_slice(ci + NBUF)], ins[b],
                                 gsems[b])
        return carry

    lax.fori_loop(0, n_chunks // NBUF, group, 0)

    for b in range(NBUF):
        pltpu.make_async_copy(
            outs[b], out_hbm.at[pl.ds(base, CHUNK), pl.ds(0, D_MODEL)],
            osems[b]).wait()


def kernel(x, table):
    b0, b1 = x.shape
    n_total = b0 * b1
    assert n_total % (NUM_WORKERS * CHUNK * NBUF) == 0
    per_w = n_total // NUM_WORKERS
    n_chunks = per_w // CHUNK

    mesh = plsc.VectorSubcoreMesh(core_axis_name="c", subcore_axis_name="s")
    emb = functools.partial(
        pl.kernel,
        mesh=mesh,
        out_type=jax.ShapeDtypeStruct((n_total, 2 * D_MODEL), jnp.float32),
        scratch_types=[
            pltpu.VMEM((per_w,), jnp.int32),
            pltpu.VMEM((CHUNK, D_MODEL), jnp.float32),
            pltpu.VMEM((CHUNK, D_MODEL), jnp.float32),
            pltpu.VMEM((CHUNK, D_MODEL), jnp.float32),
            pltpu.VMEM((CHUNK, D_MODEL), jnp.float32),
            pltpu.SemaphoreType.DMA,
            pltpu.SemaphoreType.DMA,
            pltpu.SemaphoreType.DMA,
            pltpu.SemaphoreType.DMA,
        ],
        compiler_params=pltpu.CompilerParams(use_tc_tiling_on_sc=False),
    )(functools.partial(_emb_body, per_w=per_w, n_chunks=n_chunks))

    out = emb(x.reshape(n_total), table)
    return out[:, :D_MODEL].reshape(b0, b1, D_MODEL)
